# Initial kernel scaffold; baseline (speedup 1.0000x reference)
import jax
import jax.numpy as jnp
from jax.experimental import pallas as pl


def kernel(x, edge_feats, edge_src, edge_dst, node_graph_ids, edge_graph_ids, W_ni1, W_nj1, W_fij1, W_node1, attn1, W_ni2, W_nj2, W_fij2, W_node2, attn2, W_aggN, b_aggN, W_aggE, b_aggE, W_m1, b_m1, W_m2, b_m2, W_m3, b_m3):
    raise NotImplementedError("write your pallas kernel here")


# speedup vs baseline: 17.3424x; 17.3424x over previous
import functools

import jax
import jax.numpy as jnp
from jax import lax
from jax.experimental import pallas as pl
from jax.experimental.pallas import tpu as pltpu
from jax.experimental.pallas import tpu_sc as plsc

N = 50000
E = 800000
D = 64
H = 4
HD = 16
G = 512
EB = 128
NEB = E // EB
NC, NS = 2, 16
NW = NC * NS
NROW = N // NS
BE = 2000
BN = 2000
F32 = jnp.float32


def _mesh():
    return plsc.VectorSubcoreMesh(core_axis_name="c", subcore_axis_name="s")


def _proj3(h, Wni, Wnj, Wnode):
    din = h.shape[1]

    def body(h_ref, wi_ref, wj_ref, wn_ref, hi_ref, hj_ref, hn_ref):
        hb = h_ref[...]
        hi_ref[...] = jnp.dot(hb, wi_ref[...], preferred_element_type=F32)
        hj_ref[...] = jnp.dot(hb, wj_ref[...], preferred_element_type=F32)
        hn_ref[...] = jnp.dot(hb, wn_ref[...], preferred_element_type=F32)

    w_spec = pl.BlockSpec((din, D), lambda i: (0, 0))
    o_spec = pl.BlockSpec((BN, D), lambda i: (i, 0))
    return pl.pallas_call(
        body,
        grid=(N // BN,),
        in_specs=[pl.BlockSpec((BN, din), lambda i: (i, 0)), w_spec, w_spec, w_spec],
        out_specs=[o_spec, o_spec, o_spec],
        out_shape=[jax.ShapeDtypeStruct((N, D), F32)] * 3,
    )(h, Wni, Wnj, Wnode)


def _fout_expl(f, hi_src, hj_dst, Wfij, attnM):
    fin = f.shape[1]

    def body(f_ref, hi_ref, hj_ref, wf_ref, am_ref, fo_ref, ex_ref):
        fo = hi_ref[...] + hj_ref[...] + jnp.dot(
            f_ref[...], wf_ref[...], preferred_element_type=F32)
        fo_ref[...] = fo
        lr = jnp.where(fo >= 0.0, fo, 0.2 * fo)
        l8 = jnp.dot(lr, am_ref[...], preferred_element_type=F32)
        ex_ref[...] = jnp.exp(jnp.clip(l8, -80.0, 80.0))

    return pl.pallas_call(
        body,
        grid=(E // BE,),
        in_specs=[
            pl.BlockSpec((BE, fin), lambda i: (i, 0)),
            pl.BlockSpec((BE, D), lambda i: (i, 0)),
            pl.BlockSpec((BE, D), lambda i: (i, 0)),
            pl.BlockSpec((fin, D), lambda i: (0, 0)),
            pl.BlockSpec((D, 8), lambda i: (0, 0)),
        ],
        out_specs=[
            pl.BlockSpec((BE, D), lambda i: (i, 0)),
            pl.BlockSpec((BE, 8), lambda i: (i, 0)),
        ],
        out_shape=[
            jax.ShapeDtypeStruct((E, D), F32),
            jax.ShapeDtypeStruct((E, 8), F32),
        ],
    )(f, hi_src, hj_dst, Wfij, attnM)


def _msg(hn_src, expl, s1_dst, PA, PB):
    def body(hn_ref, ex_ref, s1_ref, pa_ref, pb_ref, ma_ref, mb_ref):
        a8 = ex_ref[...] / s1_ref[...]
        arepA = jnp.dot(a8, pa_ref[...], preferred_element_type=F32)
        arepB = jnp.dot(a8, pb_ref[...], preferred_element_type=F32)
        hn = hn_ref[...]
        ma_ref[...] = hn[:, :32] * arepA
        mb_ref[...] = hn[:, 32:] * arepB

    half_o = pl.BlockSpec((BE, 32), lambda i: (i, 0))
    return pl.pallas_call(
        body,
        grid=(E // BE,),
        in_specs=[
            pl.BlockSpec((BE, D), lambda i: (i, 0)),
            pl.BlockSpec((BE, 8), lambda i: (i, 0)),
            pl.BlockSpec((BE, 8), lambda i: (i, 0)),
            pl.BlockSpec((8, 32), lambda i: (0, 0)),
            pl.BlockSpec((8, 32), lambda i: (0, 0)),
        ],
        out_specs=[half_o, half_o],
        out_shape=[jax.ShapeDtypeStruct((E, 32), F32)] * 2,
    )(hn_src, expl, s1_dst, PA, PB)


def _pool(feat, W, b2d, ids3, nblk):
    M, din = feat.shape
    bm = M // nblk

    def body(x_ref, w_ref, b_ref, id_ref, out_ref):
        i = pl.program_id(0)
        ft = jnp.dot(x_ref[...], w_ref[...], preferred_element_type=F32) + b_ref[...]
        ids = jnp.reshape(id_ref[0, 0, :], (bm, 1))
        oh = (ids == lax.broadcasted_iota(jnp.int32, (bm, G), 1)).astype(F32)
        part = lax.dot_general(oh, ft, (((0,), (0,)), ((), ())),
                               preferred_element_type=F32)

        @pl.when(i == 0)
        def _():
            out_ref[...] = jnp.zeros_like(out_ref)

        out_ref[...] += part

    return pl.pallas_call(
        body,
        grid=(nblk,),
        in_specs=[
            pl.BlockSpec((bm, din), lambda i: (i, 0)),
            pl.BlockSpec((din, HD), lambda i: (0, 0)),
            pl.BlockSpec((1, HD), lambda i: (0, 0)),
            pl.BlockSpec((1, 1, bm), lambda i: (i, 0, 0)),
        ],
        out_specs=pl.BlockSpec((G, HD), lambda i: (0, 0)),
        out_shape=jax.ShapeDtypeStruct((G, HD), F32),
    )(feat, W, b2d, ids3)


def _mlp(gn, ge, W1, b1, W2, b2, W3p, b3p):
    def body(gn_ref, ge_ref, w1_ref, b1_ref, w2_ref, b2_ref, w3_ref, b3_ref,
             out_ref):
        Gm = jnp.concatenate([gn_ref[...], ge_ref[...]], axis=1)
        z = jnp.maximum(jnp.dot(Gm, w1_ref[...], preferred_element_type=F32)
                        + b1_ref[...], 0.0)
        z = jnp.maximum(jnp.dot(z, w2_ref[...], preferred_element_type=F32)
                        + b2_ref[...], 0.0)
        out_ref[...] = jnp.dot(z, w3_ref[...], preferred_element_type=F32) \
            + b3_ref[...]

    return pl.pallas_call(
        body,
        out_shape=jax.ShapeDtypeStruct((G, 8), F32),
    )(gn, ge, W1, b1, W2, b2, W3p, b3p)


def _sc_gather2(tabA, idxA, tabB, idxB):
    dA = tabA.shape[1]
    dB = tabB.shape[1]

    @functools.partial(
        pl.kernel,
        out_type=[jax.ShapeDtypeStruct((E, dA), F32),
                  jax.ShapeDtypeStruct((E, dB), F32)],
        mesh=_mesh(),
        compiler_params=pltpu.CompilerParams(use_tc_tiling_on_sc=False),
        scratch_types=[
            pltpu.VMEM((EB,), jnp.int32), pltpu.VMEM((EB, dA), F32),
            pltpu.VMEM((EB,), jnp.int32), pltpu.VMEM((EB, dB), F32),
            pltpu.SemaphoreType.DMA, pltpu.SemaphoreType.DMA,
        ],
    )
    def k(ta_h, ia_h, tb_h, ib_h, oa_h, ob_h, iva, bufa, ivb, bufb, sa, sb):
        wid = lax.axis_index("s") * NC + lax.axis_index("c")
        trips = (NEB - wid + NW - 1) // NW

        def body(i, carry):
            off = pl.multiple_of((wid + i * NW) * EB, EB)
            pltpu.sync_copy(ia_h.at[pl.ds(off, EB)], iva)
            pltpu.sync_copy(ib_h.at[pl.ds(off, EB)], ivb)
            ca = pltpu.async_copy(ta_h.at[iva], bufa, sa)
            cb = pltpu.async_copy(tb_h.at[ivb], bufb, sb)
            ca.wait()
            cb.wait()
            pltpu.sync_copy(bufa, oa_h.at[pl.ds(off, EB)])
            pltpu.sync_copy(bufb, ob_h.at[pl.ds(off, EB)])
            return carry

        lax.fori_loop(0, trips, body, 0)

    return k(tabA, idxA, tabB, idxB)


def _sc_scatter_s1(expl, dst_idx, zeros8):
    @functools.partial(
        pl.kernel,
        out_type=jax.ShapeDtypeStruct((N, 8), F32),
        mesh=_mesh(),
        compiler_params=pltpu.CompilerParams(use_tc_tiling_on_sc=False),
        scratch_types=[
            pltpu.VMEM((EB,), jnp.int32), pltpu.VMEM((EB, 8), F32),
            pltpu.VMEM_SHARED((N, 8), F32),
        ],
    )
    def k(ex_h, di_h, z_h, s1_h, iv, buf, acc):
        cid = lax.axis_index("c")
        sid = lax.axis_index("s")

        @pl.when(cid == 0)
        def _():
            r0 = pl.multiple_of(sid * NROW, NROW)
            pltpu.sync_copy(z_h.at[pl.ds(r0, NROW)], acc.at[pl.ds(r0, NROW)])
            plsc.subcore_barrier()
            trips = (NEB - sid + NS - 1) // NS

            def body(i, carry):
                off = pl.multiple_of((sid + i * NS) * EB, EB)
                pltpu.sync_copy(di_h.at[pl.ds(off, EB)], iv)
                pltpu.sync_copy(ex_h.at[pl.ds(off, EB)], buf)
                pltpu.sync_copy(buf, acc.at[iv], add=True)
                return carry

            lax.fori_loop(0, trips, body, 0)
            plsc.subcore_barrier()
            pltpu.sync_copy(acc.at[pl.ds(r0, NROW)], s1_h.at[pl.ds(r0, NROW)])

    return k(expl, dst_idx, zeros8)


def _sc_scatter_msg(msgA, msgB, dst_idx, zeros32):
    @functools.partial(
        pl.kernel,
        out_type=[jax.ShapeDtypeStruct((N, 32), F32)] * 2,
        mesh=_mesh(),
        compiler_params=pltpu.CompilerParams(use_tc_tiling_on_sc=False),
        scratch_types=[
            pltpu.VMEM((EB,), jnp.int32), pltpu.VMEM((EB, 32), F32),
            pltpu.VMEM_SHARED((N, 32), F32),
        ],
    )
    def k(ma_h, mb_h, di_h, z_h, ha_h, hb_h, iv, buf, acc):
        cid = lax.axis_index("c")
        sid = lax.axis_index("s")
        r0 = pl.multiple_of(sid * NROW, NROW)
        pltpu.sync_copy(z_h.at[pl.ds(r0, NROW)], acc.at[pl.ds(r0, NROW)])
        plsc.subcore_barrier()
        trips = (NEB - sid + NS - 1) // NS

        def body(i, carry):
            off = pl.multiple_of((sid + i * NS) * EB, EB)
            pltpu.sync_copy(di_h.at[pl.ds(off, EB)], iv)

            @pl.when(cid == 0)
            def _():
                pltpu.sync_copy(ma_h.at[pl.ds(off, EB)], buf)

            @pl.when(cid == 1)
            def _():
                pltpu.sync_copy(mb_h.at[pl.ds(off, EB)], buf)

            pltpu.sync_copy(buf, acc.at[iv], add=True)
            return carry

        lax.fori_loop(0, trips, body, 0)
        plsc.subcore_barrier()

        @pl.when(cid == 0)
        def _():
            pltpu.sync_copy(acc.at[pl.ds(r0, NROW)], ha_h.at[pl.ds(r0, NROW)])

        @pl.when(cid == 1)
        def _():
            pltpu.sync_copy(acc.at[pl.ds(r0, NROW)], hb_h.at[pl.ds(r0, NROW)])

    return k(msgA, msgB, dst_idx, zeros32)


def _attn_mat(attn):
    cols = []
    for h_ in range(H):
        cols.append(jnp.zeros((D,), F32).at[h_ * HD:(h_ + 1) * HD].set(attn[h_]))
    cols += [jnp.zeros((D,), F32)] * 4
    return jnp.stack(cols, axis=1)


def _head_expand_mats():
    pa = jnp.zeros((8, 32), F32)
    pb = jnp.zeros((8, 32), F32)
    for h_ in range(2):
        pa = pa.at[h_, h_ * HD:(h_ + 1) * HD].set(1.0)
        pb = pb.at[2 + h_, h_ * HD:(h_ + 1) * HD].set(1.0)
    return pa, pb


def kernel(x, edge_feats, edge_src, edge_dst, node_graph_ids, edge_graph_ids,
           W_ni1, W_nj1, W_fij1, W_node1, attn1, W_ni2, W_nj2, W_fij2, W_node2,
           attn2, W_aggN, b_aggN, W_aggE, b_aggE, W_m1, b_m1, W_m2, b_m2, W_m3,
           b_m3):
    z8 = jnp.zeros((N, 8), F32)
    z32 = jnp.zeros((N, 32), F32)
    am1 = _attn_mat(attn1)
    am2 = _attn_mat(attn2)
    PA, PB = _head_expand_mats()
    ngid3 = jnp.reshape(node_graph_ids, (N // BN, 1, BN))
    egid3 = jnp.reshape(edge_graph_ids, (E // BE, 1, BE))

    h = x
    f = edge_feats
    for layer in range(3):
        if layer == 0:
            Wni, Wnj, Wfij, Wnode, am = W_ni1, W_nj1, W_fij1, W_node1, am1
        else:
            Wni, Wnj, Wfij, Wnode, am = W_ni2, W_nj2, W_fij2, W_node2, am2
        hi, hj, hn = _proj3(h, Wni, Wnj, Wnode)
        hi_src, hj_dst = _sc_gather2(hi, edge_src, hj, edge_dst)
        f, expl = _fout_expl(f, hi_src, hj_dst, Wfij, am)
        s1 = _sc_scatter_s1(expl, edge_dst, z8)
        hn_src, s1_dst = _sc_gather2(hn, edge_src, s1, edge_dst)
        ma, mb = _msg(hn_src, expl, s1_dst, PA, PB)
        ha, hb = _sc_scatter_msg(ma, mb, edge_dst, z32)
        h = jnp.concatenate([ha, hb], axis=1)

    gn = _pool(h, W_aggN, jnp.reshape(b_aggN, (1, HD)), ngid3, N // BN)
    ge = _pool(f, W_aggE, jnp.reshape(b_aggE, (1, HD)), egid3, E // BE)
    W3p = jnp.zeros((HD, 8), F32).at[:, :1].set(W_m3)
    b3p = jnp.zeros((1, 8), F32).at[:, :1].set(jnp.reshape(b_m3, (1, 1)))
    out = _mlp(gn, ge, W_m1, jnp.reshape(b_m1, (1, HD)), W_m2,
               jnp.reshape(b_m2, (1, HD)), W3p, b3p)
    return out[:, :1]
